# baseline (speedup 1.0000x reference)
import jax
import jax.numpy as jnp
from jax import lax
from jax.experimental import pallas as pl
from jax.experimental.pallas import tpu as pltpu
from jax.experimental.pallas import tpu_sc as plsc

N = 1_000_000
LANES = 16
NCORES = 2
NSUB = 16
NW = NCORES * NSUB
CHUNK = 31_248
NVREG = CHUNK // LANES
BLK = 63
NBLK = NVREG // BLK
BLK_E = BLK * LANES
TAIL_BASE = NW * CHUNK
TAIL_TILES = (N - TAIL_BASE) // LANES
SLICE_BLOCKS = (4, 9, 9, 9)
INNER_V = 63
NACC = 6
PACK = 3 * LANES

I32_MAX = 2**31 - 1
SIGN_MASK = 0x7FFFFFFF


def _sc_body(in_hbm, p_hbm, out_k, out_i, out_v,
             chunk, pvm, bmref, pd, pi, pub, sh, allp,
             sem0, sem1, sem2, sem3, sem4):
    cid = lax.axis_index("c")
    sid = lax.axis_index("s")
    wid = sid * NCORES + cid
    base = wid * CHUNK
    iota = lax.iota(jnp.int32, LANES)
    maxv = jnp.full((LANES,), I32_MAX, jnp.int32)

    pltpu.sync_copy(p_hbm, pvm)

    sems = (sem0, sem1, sem2, sem3)
    copies = []
    off = 0
    for q, nb in enumerate(SLICE_BLOCKS):
        ne = nb * BLK_E
        copies.append(pltpu.async_copy(
            in_hbm.at[pl.ds(base + off, ne)], chunk.at[pl.ds(off, ne)],
            sems[q]))
        off += ne
    has_tail = wid < TAIL_TILES
    e_off = TAIL_BASE + jnp.where(has_tail, wid, 0) * LANES
    tail_copy = pltpu.async_copy(
        in_hbm.at[pl.ds(e_off, LANES)], chunk.at[pl.ds(CHUNK, LANES)], sem4)

    pvec = pvm[...]

    def keys_of(x):
        return lax.bitcast_convert_type(x - pvec, jnp.int32) & SIGN_MASK

    sb = 0
    for q, nb in enumerate(SLICE_BLOCKS):
        copies[q].wait()

        def blk(b, _, sb=sb):
            eoff = (b + sb) * BLK_E

            def inner(t, accs):
                o2 = eoff + t * (INNER_V * LANES)
                accs = list(accs)
                for v in range(INNER_V):
                    xi = keys_of(chunk[pl.ds(o2 + v * LANES, LANES)])
                    accs[v % NACC] = jnp.minimum(accs[v % NACC], xi)
                return tuple(accs)

            accs = list(lax.fori_loop(0, BLK // INNER_V, inner,
                                      (maxv,) * NACC))
            while len(accs) > 1:
                accs = [jnp.minimum(a, b2) for a, b2 in zip(accs[::2], accs[1::2])] \
                    + ([accs[-1]] if len(accs) % 2 else [])
            bmref[pl.ds((b + sb) * LANES, LANES)] = accs[0]
            return 0

        lax.fori_loop(0, nb, blk, 0)
        sb += nb

    def bfly_min(v):
        for s in (8, 4, 2, 1):
            pd[...] = v
            v = jnp.minimum(v, plsc.load_gather(pd, [(iota + s) & (LANES - 1)]))
        return v

    def bfly_lex(k, i):
        for s in (8, 4, 2, 1):
            pd[...] = k
            pi[...] = i
            perm = (iota + s) & (LANES - 1)
            rk = plsc.load_gather(pd, [perm])
            ri = plsc.load_gather(pi, [perm])
            better = (rk < k) | ((rk == k) & (ri < i))
            k = jnp.where(better, rk, k)
            i = jnp.where(better, ri, i)
        return k, i

    macc = maxv
    for r in range(NBLK):
        macc = jnp.minimum(macc, bmref[pl.ds(r * LANES, LANES)])
    m_all = bfly_min(macc)

    bb = maxv
    for r in range(NBLK):
        hit = bmref[pl.ds(r * LANES, LANES)] == m_all
        bb = jnp.minimum(bb, jnp.where(hit, jnp.full((LANES,), r, jnp.int32),
                                       maxv))
    b_all = bfly_min(bb)

    bi = maxv
    addr0 = b_all * BLK_E + iota
    for t in range(BLK):
        av = addr0 + t * LANES
        xi = keys_of(plsc.load_gather(chunk, [av]))
        bi = jnp.minimum(bi, jnp.where(xi == m_all, av, maxv))
    i_all = bfly_min(bi)

    tail_copy.wait()
    kt = keys_of(chunk[pl.ds(CHUNK, LANES)])
    kt = jnp.where(jnp.full((LANES,), has_tail), kt, maxv)
    it = jnp.full((LANES,), CHUNK, jnp.int32) + iota
    better = (kt < m_all) | ((kt == m_all) & (it < i_all))
    fk = jnp.where(better, kt, m_all)
    fi = jnp.where(better, it, i_all)
    fk, fi = bfly_lex(fk, fi)

    fv = plsc.load_gather(chunk, [fi])
    in_main = fi < CHUNK
    fg = jnp.where(in_main, fi + base, fi - CHUNK + (TAIL_BASE + wid * LANES))

    pub[pl.ds(0, LANES)] = lax.bitcast_convert_type(fk, jnp.float32)
    pub[pl.ds(LANES, LANES)] = fg.astype(jnp.float32)
    pub[pl.ds(2 * LANES, LANES)] = fv
    pltpu.sync_copy(pub, sh.at[pl.ds(sid * PACK, PACK)])
    plsc.subcore_barrier()

    @pl.when(sid == 0)
    def _():
        pltpu.sync_copy(sh, allp)
        gd = allp[pl.ds(0, LANES)]
        gi = allp[pl.ds(LANES, LANES)]
        gv = allp[pl.ds(2 * LANES, LANES)]
        for r in range(1, NSUB):
            rd = allp[pl.ds(r * PACK, LANES)]
            ri = allp[pl.ds(r * PACK + LANES, LANES)]
            rv = allp[pl.ds(r * PACK + 2 * LANES, LANES)]
            better = (rd < gd) | ((rd == gd) & (ri < gi))
            gd = jnp.where(better, rd, gd)
            gi = jnp.where(better, ri, gi)
            gv = jnp.where(better, rv, gv)
        pub[pl.ds(0, LANES)] = gd
        pub[pl.ds(LANES, LANES)] = gi
        pub[pl.ds(2 * LANES, LANES)] = gv
        pltpu.sync_copy(pub.at[pl.ds(0, LANES)],
                        out_k.at[pl.ds(cid * LANES, LANES)])
        pltpu.sync_copy(pub.at[pl.ds(LANES, LANES)],
                        out_i.at[pl.ds(cid * LANES, LANES)])
        pltpu.sync_copy(pub.at[pl.ds(2 * LANES, LANES)],
                        out_v.at[pl.ds(cid * LANES, LANES)])


def _merge_body(k_ref, i_ref, v_ref, o_ref):
    def step(t, carry):
        bk, bi, bv = carry
        kt = k_ref[t]
        it = i_ref[t]
        vt = v_ref[t]
        better = (kt < bk) | ((kt == bk) & (it < bi))
        return (jnp.where(better, kt, bk), jnp.where(better, it, bi),
                jnp.where(better, vt, bv))

    inf = jnp.float32(jnp.inf)
    _, _, bv = lax.fori_loop(0, NCORES * LANES, step,
                             (inf, inf, jnp.float32(0.0)))
    o_ref[0] = bv


@jax.jit
def _closest_sc(inp, p16):
    mesh = plsc.VectorSubcoreMesh(
        core_axis_name="c", subcore_axis_name="s", num_cores=NCORES)
    f = pl.kernel(
        _sc_body,
        out_type=(jax.ShapeDtypeStruct((NCORES * LANES,), jnp.float32),
                  jax.ShapeDtypeStruct((NCORES * LANES,), jnp.float32),
                  jax.ShapeDtypeStruct((NCORES * LANES,), jnp.float32)),
        mesh=mesh,
        compiler_params=pltpu.CompilerParams(
            needs_layout_passes=False, use_tc_tiling_on_sc=False),
        scratch_types=[
            pltpu.VMEM((CHUNK + LANES,), jnp.float32),
            pltpu.VMEM((LANES,), jnp.float32),
            pltpu.VMEM((NBLK * LANES,), jnp.int32),
            pltpu.VMEM((LANES,), jnp.int32),
            pltpu.VMEM((LANES,), jnp.int32),
            pltpu.VMEM((PACK,), jnp.float32),
            pltpu.VMEM_SHARED((NSUB * PACK,), jnp.float32),
            pltpu.VMEM((NSUB * PACK,), jnp.float32),
            pltpu.SemaphoreType.DMA,
            pltpu.SemaphoreType.DMA,
            pltpu.SemaphoreType.DMA,
            pltpu.SemaphoreType.DMA,
            pltpu.SemaphoreType.DMA,
        ],
    )
    ck, ci, cv = f(inp, p16)
    merge = pl.pallas_call(
        _merge_body,
        out_shape=jax.ShapeDtypeStruct((1,), jnp.float32),
        in_specs=[pl.BlockSpec(memory_space=pltpu.SMEM)] * 3,
        out_specs=pl.BlockSpec(memory_space=pltpu.SMEM),
    )
    return merge(ck, ci, cv)


def kernel(input, prev_output):
    p16 = jnp.broadcast_to(prev_output, (LANES,))
    return _closest_sc(input, p16)


# speedup vs baseline: 1.0176x; 1.0176x over previous
import jax
import jax.numpy as jnp
from jax import lax
from jax.experimental import pallas as pl
from jax.experimental.pallas import tpu as pltpu
from jax.experimental.pallas import tpu_sc as plsc

N = 1_000_000
LANES = 16
NCORES = 2
NSUB = 16
NW = NCORES * NSUB
CHUNK = 31_248
NVREG = CHUNK // LANES
BLK = 63
NBLK = NVREG // BLK
BLK_E = BLK * LANES
TAIL_BASE = NW * CHUNK
TAIL_TILES = (N - TAIL_BASE) // LANES
SLICE_BLOCKS = (8, 8, 8, 7)
INNER_V = 21
NACC = 6
PACK = 3 * LANES

I32_MAX = 2**31 - 1
SIGN_MASK = 0x7FFFFFFF


def _sc_body(in_hbm, p_hbm, out_k, out_i, out_v,
             chunk, pvm, bmref, pd, pi, pub, sh, allp,
             sem0, sem1, sem2, sem3, sem4):
    cid = lax.axis_index("c")
    sid = lax.axis_index("s")
    wid = sid * NCORES + cid
    base = wid * CHUNK
    iota = lax.iota(jnp.int32, LANES)
    maxv = jnp.full((LANES,), I32_MAX, jnp.int32)

    pltpu.sync_copy(p_hbm, pvm)

    sems = (sem0, sem1, sem2, sem3)
    copies = []
    off = 0
    for q, nb in enumerate(SLICE_BLOCKS):
        ne = nb * BLK_E
        copies.append(pltpu.async_copy(
            in_hbm.at[pl.ds(base + off, ne)], chunk.at[pl.ds(off, ne)],
            sems[q]))
        off += ne
    has_tail = wid < TAIL_TILES
    e_off = TAIL_BASE + jnp.where(has_tail, wid, 0) * LANES
    tail_copy = pltpu.async_copy(
        in_hbm.at[pl.ds(e_off, LANES)], chunk.at[pl.ds(CHUNK, LANES)], sem4)

    pvec = pvm[...]

    def keys_of(x):
        return lax.bitcast_convert_type(x - pvec, jnp.int32) & SIGN_MASK

    sb = 0
    for q, nb in enumerate(SLICE_BLOCKS):
        copies[q].wait()

        def blk(b, _, sb=sb):
            eoff = (b + sb) * BLK_E

            def inner(t, accs):
                o2 = eoff + t * (INNER_V * LANES)
                accs = list(accs)
                for v in range(INNER_V):
                    xi = keys_of(chunk[pl.ds(o2 + v * LANES, LANES)])
                    accs[v % NACC] = jnp.minimum(accs[v % NACC], xi)
                return tuple(accs)

            accs = list(lax.fori_loop(0, BLK // INNER_V, inner,
                                      (maxv,) * NACC))
            while len(accs) > 1:
                accs = [jnp.minimum(a, b2) for a, b2 in zip(accs[::2], accs[1::2])] \
                    + ([accs[-1]] if len(accs) % 2 else [])
            bmref[pl.ds((b + sb) * LANES, LANES)] = accs[0]
            return 0

        lax.fori_loop(0, nb, blk, 0)
        sb += nb

    def bfly_min(v):
        for s in (8, 4, 2, 1):
            pd[...] = v
            v = jnp.minimum(v, plsc.load_gather(pd, [(iota + s) & (LANES - 1)]))
        return v

    def bfly_lex(k, i):
        for s in (8, 4, 2, 1):
            pd[...] = k
            pi[...] = i
            perm = (iota + s) & (LANES - 1)
            rk = plsc.load_gather(pd, [perm])
            ri = plsc.load_gather(pi, [perm])
            better = (rk < k) | ((rk == k) & (ri < i))
            k = jnp.where(better, rk, k)
            i = jnp.where(better, ri, i)
        return k, i

    macc = maxv
    for r in range(NBLK):
        macc = jnp.minimum(macc, bmref[pl.ds(r * LANES, LANES)])
    m_all = bfly_min(macc)

    bb = maxv
    for r in range(NBLK):
        hit = bmref[pl.ds(r * LANES, LANES)] == m_all
        bb = jnp.minimum(bb, jnp.where(hit, jnp.full((LANES,), r, jnp.int32),
                                       maxv))
    b_all = bfly_min(bb)

    bi = maxv
    addr0 = b_all * BLK_E + iota
    for t in range(BLK):
        av = addr0 + t * LANES
        xi = keys_of(plsc.load_gather(chunk, [av]))
        bi = jnp.minimum(bi, jnp.where(xi == m_all, av, maxv))
    i_all = bfly_min(bi)

    tail_copy.wait()
    kt = keys_of(chunk[pl.ds(CHUNK, LANES)])
    kt = jnp.where(jnp.full((LANES,), has_tail), kt, maxv)
    it = jnp.full((LANES,), CHUNK, jnp.int32) + iota
    better = (kt < m_all) | ((kt == m_all) & (it < i_all))
    fk = jnp.where(better, kt, m_all)
    fi = jnp.where(better, it, i_all)
    fk, fi = bfly_lex(fk, fi)

    fv = plsc.load_gather(chunk, [fi])
    in_main = fi < CHUNK
    fg = jnp.where(in_main, fi + base, fi - CHUNK + (TAIL_BASE + wid * LANES))

    pub[pl.ds(0, LANES)] = lax.bitcast_convert_type(fk, jnp.float32)
    pub[pl.ds(LANES, LANES)] = fg.astype(jnp.float32)
    pub[pl.ds(2 * LANES, LANES)] = fv
    pltpu.sync_copy(pub, sh.at[pl.ds(sid * PACK, PACK)])
    plsc.subcore_barrier()

    @pl.when(sid == 0)
    def _():
        pltpu.sync_copy(sh, allp)
        gd = allp[pl.ds(0, LANES)]
        gi = allp[pl.ds(LANES, LANES)]
        gv = allp[pl.ds(2 * LANES, LANES)]
        for r in range(1, NSUB):
            rd = allp[pl.ds(r * PACK, LANES)]
            ri = allp[pl.ds(r * PACK + LANES, LANES)]
            rv = allp[pl.ds(r * PACK + 2 * LANES, LANES)]
            better = (rd < gd) | ((rd == gd) & (ri < gi))
            gd = jnp.where(better, rd, gd)
            gi = jnp.where(better, ri, gi)
            gv = jnp.where(better, rv, gv)
        pub[pl.ds(0, LANES)] = gd
        pub[pl.ds(LANES, LANES)] = gi
        pub[pl.ds(2 * LANES, LANES)] = gv
        pltpu.sync_copy(pub.at[pl.ds(0, LANES)],
                        out_k.at[pl.ds(cid * LANES, LANES)])
        pltpu.sync_copy(pub.at[pl.ds(LANES, LANES)],
                        out_i.at[pl.ds(cid * LANES, LANES)])
        pltpu.sync_copy(pub.at[pl.ds(2 * LANES, LANES)],
                        out_v.at[pl.ds(cid * LANES, LANES)])


def _merge_body(k_ref, i_ref, v_ref, o_ref):
    def step(t, carry):
        bk, bi, bv = carry
        kt = k_ref[t]
        it = i_ref[t]
        vt = v_ref[t]
        better = (kt < bk) | ((kt == bk) & (it < bi))
        return (jnp.where(better, kt, bk), jnp.where(better, it, bi),
                jnp.where(better, vt, bv))

    inf = jnp.float32(jnp.inf)
    _, _, bv = lax.fori_loop(0, NCORES * LANES, step,
                             (inf, inf, jnp.float32(0.0)))
    o_ref[0] = bv


@jax.jit
def _closest_sc(inp, p16):
    mesh = plsc.VectorSubcoreMesh(
        core_axis_name="c", subcore_axis_name="s", num_cores=NCORES)
    f = pl.kernel(
        _sc_body,
        out_type=(jax.ShapeDtypeStruct((NCORES * LANES,), jnp.float32),
                  jax.ShapeDtypeStruct((NCORES * LANES,), jnp.float32),
                  jax.ShapeDtypeStruct((NCORES * LANES,), jnp.float32)),
        mesh=mesh,
        compiler_params=pltpu.CompilerParams(
            needs_layout_passes=False, use_tc_tiling_on_sc=False),
        scratch_types=[
            pltpu.VMEM((CHUNK + LANES,), jnp.float32),
            pltpu.VMEM((LANES,), jnp.float32),
            pltpu.VMEM((NBLK * LANES,), jnp.int32),
            pltpu.VMEM((LANES,), jnp.int32),
            pltpu.VMEM((LANES,), jnp.int32),
            pltpu.VMEM((PACK,), jnp.float32),
            pltpu.VMEM_SHARED((NSUB * PACK,), jnp.float32),
            pltpu.VMEM((NSUB * PACK,), jnp.float32),
            pltpu.SemaphoreType.DMA,
            pltpu.SemaphoreType.DMA,
            pltpu.SemaphoreType.DMA,
            pltpu.SemaphoreType.DMA,
            pltpu.SemaphoreType.DMA,
        ],
    )
    ck, ci, cv = f(inp, p16)
    merge = pl.pallas_call(
        _merge_body,
        out_shape=jax.ShapeDtypeStruct((1,), jnp.float32),
        in_specs=[pl.BlockSpec(memory_space=pltpu.SMEM)] * 3,
        out_specs=pl.BlockSpec(memory_space=pltpu.SMEM),
    )
    return merge(ck, ci, cv)


def kernel(input, prev_output):
    p16 = jnp.broadcast_to(prev_output, (LANES,))
    return _closest_sc(input, p16)
